# baseline (speedup 1.0000x reference)
import functools

import jax
import jax.numpy as jnp
from jax import lax
from jax.experimental import pallas as pl
from jax.experimental.pallas import tpu as pltpu
from jax.experimental.pallas import tpu_sc as plsc

N = 10000
E = 320000
NFEAT = 128
NHID = 64
NCLASS = 32
ALPHA = 0.2
EPS = 1e-16

NC = 2
NS = 16
NW = NC * NS
EPT = E // NW
VPT = EPT // 16
NP = 10240
NR = NP // 16
RPT = NR // NS
SL = NP // NS
BE = 128
NBT = 79
EPT2 = NBT * BE
GB = 39
TC_BLK = 512


@functools.lru_cache(maxsize=None)
def _get_mesh():
    return plsc.VectorSubcoreMesh(core_axis_name="c", subcore_axis_name="s",
                                  num_cores=NC, num_subcores=NS)


def _sc_params():
    return pltpu.CompilerParams(needs_layout_passes=False,
                                use_tc_tiling_on_sc=False)


def _mm_scores_body(x_ref, w_ref, amat_ref, wh_ref, s_ref):
    wh = jnp.dot(x_ref[...], w_ref[...], preferred_element_type=jnp.float32)
    wh_ref[...] = wh
    s_ref[...] = jnp.dot(wh, amat_ref[...], preferred_element_type=jnp.float32)


def _tc_mm_scores(x, w, amat):
    fin, h = w.shape
    grid = (NP // TC_BLK,)
    return pl.pallas_call(
        _mm_scores_body,
        grid=grid,
        in_specs=[
            pl.BlockSpec((TC_BLK, fin), lambda i: (i, 0)),
            pl.BlockSpec((fin, h), lambda i: (0, 0)),
            pl.BlockSpec((h, 2), lambda i: (0, 0)),
        ],
        out_specs=[
            pl.BlockSpec((TC_BLK, h), lambda i: (i, 0)),
            pl.BlockSpec((TC_BLK, 2), lambda i: (i, 0)),
        ],
        out_shape=[
            jax.ShapeDtypeStruct((NP, h), jnp.float32),
            jax.ShapeDtypeStruct((NP, 2), jnp.float32),
        ],
    )(x, w, amat)


def _combine_mm_body(u0_ref, u1_ref, s0_ref, s1_ref, w_ref, amat_ref,
                     wh_ref, s_ref):
    den = s0_ref[...] + s1_ref[...] + EPS
    hx = (u0_ref[...] + u1_ref[...]) / den
    hx = jnp.where(hx >= 0, hx, ALPHA * hx)
    wh = jnp.dot(hx, w_ref[...], preferred_element_type=jnp.float32)
    wh_ref[...] = wh
    s_ref[...] = jnp.dot(wh, amat_ref[...], preferred_element_type=jnp.float32)


def _tc_combine_mm(u0, u1, s0, s1, w, amat):
    fin, h = w.shape
    grid = (NP // TC_BLK,)
    return pl.pallas_call(
        _combine_mm_body,
        grid=grid,
        in_specs=[
            pl.BlockSpec((TC_BLK, fin), lambda i: (i, 0)),
            pl.BlockSpec((TC_BLK, fin), lambda i: (i, 0)),
            pl.BlockSpec((TC_BLK, 1), lambda i: (i, 0)),
            pl.BlockSpec((TC_BLK, 1), lambda i: (i, 0)),
            pl.BlockSpec((fin, h), lambda i: (0, 0)),
            pl.BlockSpec((h, 2), lambda i: (0, 0)),
        ],
        out_specs=[
            pl.BlockSpec((TC_BLK, h), lambda i: (i, 0)),
            pl.BlockSpec((TC_BLK, 2), lambda i: (i, 0)),
        ],
        out_shape=[
            jax.ShapeDtypeStruct((NP, h), jnp.float32),
            jax.ShapeDtypeStruct((NP, 2), jnp.float32),
        ],
    )(u0, u1, s0, s1, w, amat)


def _final_body(u0_ref, u1_ref, s0_ref, s1_ref, o_ref):
    den = s0_ref[...] + s1_ref[...] + EPS
    o_ref[...] = (u0_ref[...] + u1_ref[...]) / den


def _tc_final(u0, u1, s0, s1):
    h = u0.shape[-1]
    grid = (NP // TC_BLK,)
    return pl.pallas_call(
        _final_body,
        grid=grid,
        in_specs=[
            pl.BlockSpec((TC_BLK, h), lambda i: (i, 0)),
            pl.BlockSpec((TC_BLK, h), lambda i: (i, 0)),
            pl.BlockSpec((TC_BLK, 1), lambda i: (i, 0)),
            pl.BlockSpec((TC_BLK, 1), lambda i: (i, 0)),
        ],
        out_specs=pl.BlockSpec((TC_BLK, h), lambda i: (i, 0)),
        out_shape=jax.ShapeDtypeStruct((NP, h), jnp.float32),
    )(u0, u1, s0, s1)


def _zero16(tracer):
    return jnp.full((16,), (tracer * 0).astype(jnp.float32))


def _fill_row_idx(idx2):
    base = lax.iota(jnp.int32, 16)
    for c in range(NR // 128):
        for k in range(8):
            idx2[c, pl.ds(k * 16, 16)] = base + (c * 128 + k * 16)


def _split_idx(ir):
    return [lax.shift_right_logical(ir, 4), lax.bitwise_and(ir, 15)]


def _reduce_tiles(loc2, sh2, idx2, zrows, out3, sid):
    pltpu.sync_copy(zrows, sh2.at[pl.ds(sid * RPT, RPT)])
    plsc.subcore_barrier()
    for c in range(NR // 128):
        pltpu.sync_copy(loc2.at[pl.ds(c * 128, 128)], sh2.at[idx2.at[c]],
                        add=True)
    plsc.subcore_barrier()
    pltpu.sync_copy(sh2.at[pl.ds(sid * RPT, RPT)], out3)


@functools.lru_cache(maxsize=None)
def _get_sc_edge_scores():
  @functools.partial(
    pl.kernel,
    out_type=(
        jax.ShapeDtypeStruct((NW, EPT2), jnp.float32),
        jax.ShapeDtypeStruct((NC, NR, 16), jnp.float32),
    ),
    mesh=_get_mesh(),
    compiler_params=_sc_params(),
    scratch_types=[
        pltpu.VMEM((EPT2,), jnp.int32),
        pltpu.VMEM((EPT2,), jnp.int32),
        pltpu.VMEM((NP,), jnp.float32),
        pltpu.VMEM((NP,), jnp.float32),
        pltpu.VMEM((EPT2,), jnp.float32),
        pltpu.VMEM((NR, 16), jnp.float32),
        pltpu.VMEM((NR // 128, 128), jnp.int32),
        pltpu.VMEM((RPT, 16), jnp.float32),
        pltpu.VMEM_SHARED((NR, 16), jnp.float32),
    ],
  )
  def _sc_edge_scores(row_f, col_f, ssrc, sdst, e2, rpart,
                      row_v, col_v, ssrc_v, sdst_v, e_v, r_loc,
                      idx2, zrows, r_sh):
    cid = lax.axis_index("c")
    sid = lax.axis_index("s")
    wid = cid * NS + sid
    pltpu.sync_copy(row_f.at[wid], row_v)
    pltpu.sync_copy(col_f.at[wid], col_v)
    pltpu.sync_copy(ssrc, ssrc_v)
    pltpu.sync_copy(sdst, sdst_v)
    _fill_row_idx(idx2)
    for zi in range(RPT):
        zrows[zi, :] = _zero16(cid)

    def zbody(i, _):
        r_loc[i, :] = _zero16(i)
        return 0

    lax.fori_loop(0, NR, zbody, 0)

    def ebody(i, _):
        ir = row_v[pl.ds(i * 16, 16)]
        ic = col_v[pl.ds(i * 16, 16)]
        z = plsc.load_gather(ssrc_v, [ir]) + plsc.load_gather(sdst_v, [ic])
        ev = jnp.where(z >= 0, z, ALPHA * z)
        e_v[pl.ds(i * 16, 16)] = ev
        plsc.addupdate_scatter(r_loc, _split_idx(ir), ev)
        return 0

    lax.fori_loop(0, VPT, ebody, 0)
    pltpu.sync_copy(e_v, e2.at[wid])
    _reduce_tiles(r_loc, r_sh, idx2, zrows,
                  rpart.at[cid, pl.ds(sid * RPT, RPT)], sid)

  return _sc_edge_scores


@functools.lru_cache(maxsize=None)
def _make_sc_aggregate(H):
    nb = H // 16

    @functools.partial(
        pl.kernel,
        out_type=(
            jax.ShapeDtypeStruct((NC, NR, 16), jnp.float32),
            jax.ShapeDtypeStruct((NC, NP, H), jnp.float32),
        ),
        mesh=_get_mesh(),
        compiler_params=_sc_params(),
        scratch_types=[
            pltpu.VMEM((NBT, BE), jnp.int32),
            pltpu.VMEM((EPT2,), jnp.int32),
            pltpu.VMEM((EPT2,), jnp.float32),
            pltpu.VMEM((NR, 16), jnp.float32),
            pltpu.VMEM((NR, 16), jnp.float32),
            pltpu.VMEM((2, BE, H), jnp.float32),
            pltpu.VMEM((BE, H), jnp.float32),
            pltpu.VMEM((64, H), jnp.float32),
            pltpu.VMEM((NR // 128, 128), jnp.int32),
            pltpu.VMEM((RPT, 16), jnp.float32),
            pltpu.SemaphoreType.DMA,
            pltpu.SemaphoreType.DMA,
            pltpu.VMEM_SHARED((NR, 16), jnp.float32),
            pltpu.VMEM_SHARED((NP, H), jnp.float32),
        ],
    )
    def _sc_aggregate(row_b, col_f, e2, rcomb, wh, spart, upart,
                      row_bv, col_v, ee_v, r_v, s_loc,
                      wh_buf, out_buf, zbuf, idx2, zrows, ssem, gsem,
                      s_sh, acc):
        cid = lax.axis_index("c")
        sid = lax.axis_index("s")
        wid = cid * NS + sid
        pltpu.sync_copy(row_b.at[wid], row_bv)
        pltpu.sync_copy(col_f.at[wid], col_v)
        pltpu.sync_copy(e2.at[wid], ee_v)
        pltpu.sync_copy(rcomb, r_v)
        _fill_row_idx(idx2)
        for zi in range(RPT):
            zrows[zi, :] = _zero16(cid)
        for t in range((EPT2 - EPT) // 16):
            ee_v[pl.ds(EPT + t * 16, 16)] = _zero16(cid)

        def zsbody(i, _):
            s_loc[i, :] = _zero16(i)
            return 0

        lax.fori_loop(0, NR, zsbody, 0)

        def zbbody(i, _):
            for k in range(nb):
                zbuf[i, pl.ds(k * 16, 16)] = _zero16(i)
            return 0

        lax.fori_loop(0, 64, zbbody, 0)
        base = sid * SL
        for z in range(SL // 64):
            pltpu.sync_copy(zbuf, acc.at[pl.ds(base + z * 64, 64)])
        plsc.subcore_barrier()

        def wbody(i, _):
            ir = row_bv[i // 8, pl.ds(lax.rem(i, 8) * 16, 16)]
            rv = plsc.load_gather(r_v, _split_idx(ir))
            ee = jnp.exp(ee_v[pl.ds(i * 16, 16)] - rv)
            ee_v[pl.ds(i * 16, 16)] = ee
            plsc.addupdate_scatter(s_loc, _split_idx(ir), ee)
            return 0

        lax.fori_loop(0, VPT, wbody, 0)

        def gidx(b):
            return col_v.at[pl.ds(b * BE, BE)]

        pltpu.async_copy(wh.at[gidx(0)], wh_buf.at[0], gsem)

        def batch(b, _):
            pb = lax.bitwise_and(b, 1)
            pltpu.make_async_copy(wh.at[gidx(b)], wh_buf.at[pb],
                                  gsem).wait()

            @pl.when(b + 1 < NBT)
            def _():
                pltpu.async_copy(wh.at[gidx(b + 1)], wh_buf.at[1 - pb],
                                 gsem)

            @pl.when(b > 0)
            def _():
                pltpu.make_async_copy(out_buf,
                                      acc.at[row_bv.at[b - 1]], ssem).wait()

            for g in range(BE // 16):
                wv = ee_v[pl.ds(b * BE + g * 16, 16)]
                for j in range(16):
                    av = jnp.full((16,), wv[j], jnp.float32)
                    r = g * 16 + j
                    for k in range(nb):
                        out_buf[r, pl.ds(k * 16, 16)] = (
                            wh_buf[pb, r, pl.ds(k * 16, 16)] * av)
            pltpu.async_copy(out_buf, acc.at[row_bv.at[b]], ssem, add=True)
            return 0

        lax.fori_loop(0, NBT, batch, 0)
        pltpu.make_async_copy(out_buf, acc.at[row_bv.at[NBT - 1]],
                              ssem).wait()

        plsc.subcore_barrier()
        _reduce_tiles(s_loc, s_sh, idx2, zrows,
                      spart.at[cid, pl.ds(sid * RPT, RPT)], sid)
        pltpu.sync_copy(acc.at[pl.ds(sid * SL, SL)],
                        upart.at[cid, pl.ds(sid * SL, SL)])

    return _sc_aggregate


def kernel(x, edge_index, W1, a1, W2, a2):
    row = edge_index[0]
    col = edge_index[1]
    row_f = jnp.pad(row.reshape(NW, EPT), ((0, 0), (0, EPT2 - EPT)),
                    constant_values=N)
    col_f = jnp.pad(col.reshape(NW, EPT), ((0, 0), (0, EPT2 - EPT)))
    row_b = row_f.reshape(NW, NBT, BE) + 0
    xp = jnp.pad(x, ((0, NP - N), (0, 0)))
    amat1 = jnp.stack([a1[:NHID, 0], a1[NHID:, 0]], axis=1)
    amat2 = jnp.stack([a2[:NCLASS, 0], a2[NCLASS:, 0]], axis=1)

    sc_edge_scores = _get_sc_edge_scores()

    wh1, s1 = _tc_mm_scores(xp, W1, amat1)
    e1, r1 = sc_edge_scores(row_f, col_f, s1[:, 0], s1[:, 1])
    sp1, up1 = _make_sc_aggregate(NHID)(row_b, col_f, e1, r1[0] + r1[1], wh1)
    sp1f = sp1.reshape(NC, NP)

    wh2, s2 = _tc_combine_mm(up1[0], up1[1], sp1f[0][:, None],
                             sp1f[1][:, None], W2, amat2)
    e2, r2 = sc_edge_scores(row_f, col_f, s2[:, 0], s2[:, 1])
    sp2, up2 = _make_sc_aggregate(NCLASS)(row_b, col_f, e2, r2[0] + r2[1], wh2)
    sp2f = sp2.reshape(NC, NP)

    out = _tc_final(up2[0], up2[1], sp2f[0][:, None], sp2f[1][:, None])
    return out[:N]


# speedup vs baseline: 1.3225x; 1.3225x over previous
import functools

import jax
import jax.numpy as jnp
from jax import lax
from jax.experimental import pallas as pl
from jax.experimental.pallas import tpu as pltpu
from jax.experimental.pallas import tpu_sc as plsc

N = 10000
E = 320000
NFEAT = 128
NHID = 64
NCLASS = 32
ALPHA = 0.2
EPS = 1e-16

NC = 2
NS = 16
NW = NC * NS
EPT = E // NW
VPT = EPT // 16
NP = 10240
NR = NP // 16
RPT = NR // NS
SL = NP // NS
BE = 128
NBT = 79
EPT2 = NBT * BE
TC_BLK = 512


@functools.lru_cache(maxsize=None)
def _get_mesh():
    return plsc.VectorSubcoreMesh(core_axis_name="c", subcore_axis_name="s",
                                  num_cores=NC, num_subcores=NS)


def _sc_params():
    return pltpu.CompilerParams(needs_layout_passes=False,
                                use_tc_tiling_on_sc=False)


def _mm_scores_body(x_ref, w_ref, amat_ref, wh_ref, s_ref):
    wh = jnp.dot(x_ref[...], w_ref[...], preferred_element_type=jnp.float32)
    wh_ref[...] = wh
    s_ref[...] = jnp.dot(wh, amat_ref[...], preferred_element_type=jnp.float32)


def _tc_mm_scores(x, w, amat):
    fin, h = w.shape
    grid = (NP // TC_BLK,)
    return pl.pallas_call(
        _mm_scores_body,
        grid=grid,
        in_specs=[
            pl.BlockSpec((TC_BLK, fin), lambda i: (i, 0)),
            pl.BlockSpec((fin, h), lambda i: (0, 0)),
            pl.BlockSpec((h, 2), lambda i: (0, 0)),
        ],
        out_specs=[
            pl.BlockSpec((TC_BLK, h), lambda i: (i, 0)),
            pl.BlockSpec((TC_BLK, 2), lambda i: (i, 0)),
        ],
        out_shape=[
            jax.ShapeDtypeStruct((NP, h), jnp.float32),
            jax.ShapeDtypeStruct((NP, 2), jnp.float32),
        ],
    )(x, w, amat)


def _combine_mm_body(u0_ref, u1_ref, s0_ref, s1_ref, w_ref, amat_ref,
                     wh_ref, s_ref):
    den = s0_ref[...] + s1_ref[...] + EPS
    hx = (u0_ref[...] + u1_ref[...]) / den
    hx = jnp.where(hx >= 0, hx, ALPHA * hx)
    wh = jnp.dot(hx, w_ref[...], preferred_element_type=jnp.float32)
    wh_ref[...] = wh
    s_ref[...] = jnp.dot(wh, amat_ref[...], preferred_element_type=jnp.float32)


def _tc_combine_mm(u0, u1, s0, s1, w, amat):
    fin, h = w.shape
    grid = (NP // TC_BLK,)
    return pl.pallas_call(
        _combine_mm_body,
        grid=grid,
        in_specs=[
            pl.BlockSpec((TC_BLK, fin), lambda i: (i, 0)),
            pl.BlockSpec((TC_BLK, fin), lambda i: (i, 0)),
            pl.BlockSpec((TC_BLK, 1), lambda i: (i, 0)),
            pl.BlockSpec((TC_BLK, 1), lambda i: (i, 0)),
            pl.BlockSpec((fin, h), lambda i: (0, 0)),
            pl.BlockSpec((h, 2), lambda i: (0, 0)),
        ],
        out_specs=[
            pl.BlockSpec((TC_BLK, h), lambda i: (i, 0)),
            pl.BlockSpec((TC_BLK, 2), lambda i: (i, 0)),
        ],
        out_shape=[
            jax.ShapeDtypeStruct((NP, h), jnp.float32),
            jax.ShapeDtypeStruct((NP, 2), jnp.float32),
        ],
    )(u0, u1, s0, s1, w, amat)


def _final_body(u0_ref, u1_ref, s0_ref, s1_ref, o_ref):
    den = s0_ref[...] + s1_ref[...] + EPS
    o_ref[...] = (u0_ref[...] + u1_ref[...]) / den


def _tc_final(u0, u1, s0, s1):
    h = u0.shape[-1]
    grid = (NP // TC_BLK,)
    return pl.pallas_call(
        _final_body,
        grid=grid,
        in_specs=[
            pl.BlockSpec((TC_BLK, h), lambda i: (i, 0)),
            pl.BlockSpec((TC_BLK, h), lambda i: (i, 0)),
            pl.BlockSpec((TC_BLK, 1), lambda i: (i, 0)),
            pl.BlockSpec((TC_BLK, 1), lambda i: (i, 0)),
        ],
        out_specs=pl.BlockSpec((TC_BLK, h), lambda i: (i, 0)),
        out_shape=jax.ShapeDtypeStruct((NP, h), jnp.float32),
    )(u0, u1, s0, s1)


def _zero16(tracer):
    return jnp.full((16,), (tracer * 0).astype(jnp.float32))


def _fill_row_idx(idx2):
    base = lax.iota(jnp.int32, 16)
    for c in range(NR // 128):
        for k in range(8):
            idx2[c, pl.ds(k * 16, 16)] = base + (c * 128 + k * 16)


def _split_idx(ir):
    return [lax.shift_right_logical(ir, 4), lax.bitwise_and(ir, 15)]


def _reduce_tiles(loc2, sh2, idx2, zrows, out3, sid):
    pltpu.sync_copy(zrows, sh2.at[pl.ds(sid * RPT, RPT)])
    plsc.subcore_barrier()
    for c in range(NR // 128):
        pltpu.sync_copy(loc2.at[pl.ds(c * 128, 128)], sh2.at[idx2.at[c]],
                        add=True)
    plsc.subcore_barrier()
    pltpu.sync_copy(sh2.at[pl.ds(sid * RPT, RPT)], out3)


@functools.lru_cache(maxsize=None)
def _get_sc_edge_scores():
  @functools.partial(
    pl.kernel,
    out_type=(
        jax.ShapeDtypeStruct((NW, EPT2), jnp.float32),
        jax.ShapeDtypeStruct((NC, NR, 16), jnp.float32),
    ),
    mesh=_get_mesh(),
    compiler_params=_sc_params(),
    scratch_types=[
        pltpu.VMEM((EPT2,), jnp.int32),
        pltpu.VMEM((EPT2,), jnp.int32),
        pltpu.VMEM((NP,), jnp.float32),
        pltpu.VMEM((NP,), jnp.float32),
        pltpu.VMEM((EPT2,), jnp.float32),
        pltpu.VMEM((NR, 16), jnp.float32),
        pltpu.VMEM((NR // 128, 128), jnp.int32),
        pltpu.VMEM((RPT, 16), jnp.float32),
        pltpu.VMEM_SHARED((NR, 16), jnp.float32),
    ],
  )
  def _sc_edge_scores(row_f, col_f, ssrc, sdst, e2, rpart,
                      row_v, col_v, ssrc_v, sdst_v, e_v, r_loc,
                      idx2, zrows, r_sh):
    cid = lax.axis_index("c")
    sid = lax.axis_index("s")
    wid = cid * NS + sid
    pltpu.sync_copy(row_f.at[wid], row_v)
    pltpu.sync_copy(col_f.at[wid], col_v)
    pltpu.sync_copy(ssrc, ssrc_v)
    pltpu.sync_copy(sdst, sdst_v)
    _fill_row_idx(idx2)
    for zi in range(RPT):
        zrows[zi, :] = _zero16(cid)

    def zbody(i, _):
        r_loc[i, :] = _zero16(i)
        return 0

    lax.fori_loop(0, NR, zbody, 0)

    def ebody(i, _):
        ir = row_v[pl.ds(i * 16, 16)]
        ic = col_v[pl.ds(i * 16, 16)]
        z = plsc.load_gather(ssrc_v, [ir]) + plsc.load_gather(sdst_v, [ic])
        ev = jnp.where(z >= 0, z, ALPHA * z)
        e_v[pl.ds(i * 16, 16)] = ev
        plsc.addupdate_scatter(r_loc, _split_idx(ir), ev)
        return 0

    lax.fori_loop(0, VPT, ebody, 0)
    pltpu.sync_copy(e_v, e2.at[wid])
    _reduce_tiles(r_loc, r_sh, idx2, zrows,
                  rpart.at[cid, pl.ds(sid * RPT, RPT)], sid)

  return _sc_edge_scores


@functools.lru_cache(maxsize=None)
def _make_sc_aggregate(H):
    nb = H // 16

    @functools.partial(
        pl.kernel,
        out_type=(
            jax.ShapeDtypeStruct((NC, NR, 16), jnp.float32),
            jax.ShapeDtypeStruct((NC, NP, H), jnp.float32),
        ),
        mesh=_get_mesh(),
        compiler_params=_sc_params(),
        scratch_types=[
            pltpu.VMEM((NBT, BE), jnp.int32),
            pltpu.VMEM((EPT2,), jnp.int32),
            pltpu.VMEM((EPT2,), jnp.float32),
            pltpu.VMEM((NR, 16), jnp.float32),
            pltpu.VMEM((NR, 16), jnp.float32),
            pltpu.VMEM((BE, H), jnp.float32),
            pltpu.VMEM((BE, H), jnp.float32),
            pltpu.VMEM((64, H), jnp.float32),
            pltpu.VMEM((NR // 128, 128), jnp.int32),
            pltpu.VMEM((RPT, 16), jnp.float32),
            pltpu.SemaphoreType.DMA,
            pltpu.VMEM_SHARED((NR, 16), jnp.float32),
            pltpu.VMEM_SHARED((NP, H), jnp.float32),
        ],
    )
    def _sc_aggregate(row_b, col_f, e2, rcomb, wh, spart, upart,
                      row_bv, col_v, ee_v, r_v, s_loc,
                      wh_buf, out_buf, zbuf, idx2, zrows, ssem, s_sh, acc):
        cid = lax.axis_index("c")
        sid = lax.axis_index("s")
        wid = cid * NS + sid
        pltpu.sync_copy(row_b.at[wid], row_bv)
        pltpu.sync_copy(col_f.at[wid], col_v)
        pltpu.sync_copy(e2.at[wid], ee_v)
        pltpu.sync_copy(rcomb, r_v)
        _fill_row_idx(idx2)
        for zi in range(RPT):
            zrows[zi, :] = _zero16(cid)
        for t in range((EPT2 - EPT) // 16):
            ee_v[pl.ds(EPT + t * 16, 16)] = _zero16(cid)

        def zsbody(i, _):
            s_loc[i, :] = _zero16(i)
            return 0

        lax.fori_loop(0, NR, zsbody, 0)

        def zbbody(i, _):
            for k in range(nb):
                zbuf[i, pl.ds(k * 16, 16)] = _zero16(i)
            return 0

        lax.fori_loop(0, 64, zbbody, 0)
        base = sid * SL
        for z in range(SL // 64):
            pltpu.sync_copy(zbuf, acc.at[pl.ds(base + z * 64, 64)])
        plsc.subcore_barrier()

        def wbody(i, _):
            ir = row_bv[i // 8, pl.ds(lax.rem(i, 8) * 16, 16)]
            rv = plsc.load_gather(r_v, _split_idx(ir))
            ee = jnp.exp(ee_v[pl.ds(i * 16, 16)] - rv)
            ee_v[pl.ds(i * 16, 16)] = ee
            plsc.addupdate_scatter(s_loc, _split_idx(ir), ee)
            return 0

        lax.fori_loop(0, VPT, wbody, 0)

        def batch(b, _):
            pltpu.sync_copy(wh.at[col_v.at[pl.ds(b * BE, BE)]], wh_buf)

            @pl.when(b > 0)
            def _():
                pltpu.make_async_copy(out_buf,
                                      acc.at[row_bv.at[b - 1]], ssem).wait()

            for g in range(BE // 16):
                wv = ee_v[pl.ds(b * BE + g * 16, 16)]
                for j in range(16):
                    av = jnp.full((16,), wv[j], jnp.float32)
                    r = g * 16 + j
                    for k in range(nb):
                        out_buf[r, pl.ds(k * 16, 16)] = (
                            wh_buf[r, pl.ds(k * 16, 16)] * av)
            pltpu.async_copy(out_buf, acc.at[row_bv.at[b]], ssem, add=True)
            return 0

        lax.fori_loop(0, NBT, batch, 0)
        pltpu.make_async_copy(out_buf, acc.at[row_bv.at[NBT - 1]],
                              ssem).wait()

        plsc.subcore_barrier()
        _reduce_tiles(s_loc, s_sh, idx2, zrows,
                      spart.at[cid, pl.ds(sid * RPT, RPT)], sid)
        pltpu.sync_copy(acc.at[pl.ds(sid * SL, SL)],
                        upart.at[cid, pl.ds(sid * SL, SL)])

    return _sc_aggregate


def kernel(x, edge_index, W1, a1, W2, a2):
    row = edge_index[0]
    col = edge_index[1]
    row_f = jnp.pad(row.reshape(NW, EPT), ((0, 0), (0, EPT2 - EPT)),
                    constant_values=N)
    col_f = jnp.pad(col.reshape(NW, EPT), ((0, 0), (0, EPT2 - EPT)))
    row_b = row_f.reshape(NW, NBT, BE) + 0
    xp = jnp.pad(x, ((0, NP - N), (0, 0)))
    amat1 = jnp.stack([a1[:NHID, 0], a1[NHID:, 0]], axis=1)
    amat2 = jnp.stack([a2[:NCLASS, 0], a2[NCLASS:, 0]], axis=1)

    sc_edge_scores = _get_sc_edge_scores()

    wh1, s1 = _tc_mm_scores(xp, W1, amat1)
    e1, r1 = sc_edge_scores(row_f, col_f, s1[:, 0], s1[:, 1])
    sp1, up1 = _make_sc_aggregate(NHID)(row_b, col_f, e1, r1[0] + r1[1], wh1)
    sp1f = sp1.reshape(NC, NP)

    wh2, s2 = _tc_combine_mm(up1[0], up1[1], sp1f[0][:, None],
                             sp1f[1][:, None], W2, amat2)
    e2, r2 = sc_edge_scores(row_f, col_f, s2[:, 0], s2[:, 1])
    sp2, up2 = _make_sc_aggregate(NCLASS)(row_b, col_f, e2, r2[0] + r2[1], wh2)
    sp2f = sp2.reshape(NC, NP)

    out = _tc_final(up2[0], up2[1], sp2f[0][:, None], sp2f[1][:, None])
    return out[:N]
